# baseline (speedup 1.0000x reference)
import math
import functools

import jax
import jax.numpy as jnp
from jax.experimental import pallas as pl

N_OP, N_MACHINE, N_JOB = 10000, 200, 500
HID, HEADS = 256, 8
DH = HID // HEADS
NODE_TYPES = ['op', 'machine', 'job']
NODE_COUNTS = {'op': N_OP, 'machine': N_MACHINE, 'job': N_JOB}
EDGE_TYPES = [('job', 'contains', 'op'), ('op', 'belongs_to', 'job'),
              ('op', 'precedes', 'op'), ('op', 'machine_precedes', 'op'),
              ('op', 'on_machine', 'machine'), ('op', 'assigned_to', 'machine'),
              ('machine', 'can_process', 'op'), ('machine', 'processes', 'op')]


def _graph_norm(x, w, b, eps=1e-5):
    mean = jnp.mean(x)
    out = x - mean
    var = jnp.mean(out * out)
    return out / jnp.sqrt(var + eps) * w + b


def _seg_softmax(alpha, dst, n):
    amax = jax.ops.segment_max(alpha, dst, num_segments=n)
    amax = jnp.where(jnp.isfinite(amax), amax, 0.0)
    e = jnp.exp(alpha - amax[dst])
    s = jax.ops.segment_sum(e, dst, num_segments=n)
    return e / (s[dst] + 1e-16)


def _hgt_layer(lp, x_dict, edges):
    kd, qd, vd = {}, {}, {}
    for nt in NODE_TYPES:
        kqv = x_dict[nt] @ lp['kqv_w_' + nt] + lp['kqv_b_' + nt]
        k, q, v = jnp.split(kqv, 3, axis=-1)
        kd[nt] = k.reshape(-1, HEADS, DH)
        qd[nt] = q.reshape(-1, HEADS, DH)
        vd[nt] = v.reshape(-1, HEADS, DH)
    agg = {nt: jnp.zeros((NODE_COUNTS[nt], HEADS, DH), dtype=jnp.float32)
           for nt in NODE_TYPES}
    for st, rel, dt in EDGE_TYPES:
        src, dst = edges[rel]
        k = jnp.einsum('nhd,hde->nhe', kd[st], lp['a_rel_' + rel])
        v = jnp.einsum('nhd,hde->nhe', vd[st], lp['m_rel_' + rel])
        alpha = jnp.sum(qd[dt][dst] * k[src], axis=-1) * lp['p_rel_' + rel] / math.sqrt(DH)
        alpha = _seg_softmax(alpha, dst, NODE_COUNTS[dt])
        msg = v[src] * alpha[:, :, None]
        agg[dt] = agg[dt] + jax.ops.segment_sum(msg, dst, num_segments=NODE_COUNTS[dt])
    out = {}
    for nt in NODE_TYPES:
        o = jax.nn.gelu(agg[nt].reshape(NODE_COUNTS[nt], HID))
        o = o @ lp['out_w_' + nt] + lp['out_b_' + nt]
        a = jax.nn.sigmoid(lp['skip_' + nt])
        out[nt] = a * o + (1.0 - a) * x_dict[nt]
    return out


def _mlp_head_kernel(a_ref, w2_ref, b2_ref, w3_ref, b3_ref, w4_ref, b4_ref,
                     out_ref):
    h = jnp.tanh(a_ref[...])
    h = jnp.tanh(h @ w2_ref[...] + b2_ref[...])
    h = jnp.tanh(h @ w3_ref[...] + b3_ref[...])
    out_ref[...] = (h @ w4_ref[...] + b4_ref[...])


def _mlp_head(a, w2, b2, w3, b3, w4, b4):
    n = a.shape[0]
    blk = 1000
    grid = (n // blk,)
    return pl.pallas_call(
        _mlp_head_kernel,
        grid=grid,
        in_specs=[
            pl.BlockSpec((blk, HID), lambda i: (i, 0)),
            pl.BlockSpec((HID, HID), lambda i: (0, 0)),
            pl.BlockSpec((HID,), lambda i: (0,)),
            pl.BlockSpec((HID, 128), lambda i: (0, 0)),
            pl.BlockSpec((128,), lambda i: (0,)),
            pl.BlockSpec((128, 1), lambda i: (0, 0)),
            pl.BlockSpec((1,), lambda i: (0,)),
        ],
        out_specs=pl.BlockSpec((blk, 1), lambda i: (i, 0)),
        out_shape=jax.ShapeDtypeStruct((n, 1), jnp.float32),
    )(a, w2, b2, w3, b3, w4, b4)


def kernel(params, op_x, machine_x, job_x, src_contains, dst_contains,
           src_belongs_to, dst_belongs_to, src_precedes, dst_precedes,
           src_machine_precedes, dst_machine_precedes, src_on_machine,
           dst_on_machine, src_assigned_to, dst_assigned_to, src_can_process,
           dst_can_process, src_processes, dst_processes, act_op, act_machine):
    edges = {'contains': (src_contains, dst_contains),
             'belongs_to': (src_belongs_to, dst_belongs_to),
             'precedes': (src_precedes, dst_precedes),
             'machine_precedes': (src_machine_precedes, dst_machine_precedes),
             'on_machine': (src_on_machine, dst_on_machine),
             'assigned_to': (src_assigned_to, dst_assigned_to),
             'can_process': (src_can_process, dst_can_process),
             'processes': (src_processes, dst_processes)}
    feats = {'op': op_x, 'machine': machine_x, 'job': job_x}
    x = {}
    for nt in NODE_TYPES:
        h = feats[nt] @ params['emb_w_' + nt] + params['emb_b_' + nt]
        x[nt] = _graph_norm(h, params['norm0_w_' + nt], params['norm0_b_' + nt])
    for lp in params['layers']:
        res = x
        out = _hgt_layer(lp, x, edges)
        x = {nt: _graph_norm(out[nt] + res[nt], lp['ln_w_' + nt], lp['ln_b_' + nt])
             for nt in NODE_TYPES}

    g = jnp.concatenate([jnp.mean(x['op'], axis=0),
                         jnp.mean(x['machine'], axis=0),
                         jnp.mean(x['job'], axis=0)])
    w1 = params['mlp_w'][0]
    p_op = x['op'] @ w1[:HID]
    p_m = x['machine'] @ w1[HID:2 * HID]
    cg = g @ w1[2 * HID:] + params['mlp_b'][0]
    a = p_op[act_op] + p_m[act_machine] + cg[None, :]
    out = _mlp_head(a, params['mlp_w'][1], params['mlp_b'][1],
                    params['mlp_w'][2], params['mlp_b'][2],
                    params['mlp_w'][3], params['mlp_b'][3])
    return out[:, 0]


# speedup vs baseline: 7.5170x; 7.5170x over previous
import math
import functools

import jax
import jax.numpy as jnp
from jax import lax
from jax.experimental import pallas as pl
from jax.experimental.pallas import tpu as pltpu, tpu_sc as plsc

N_OP, N_MACHINE, N_JOB = 10000, 200, 500
HID, HEADS = 256, 8
DH = HID // HEADS
NODE_TYPES = ['op', 'machine', 'job']
NODE_COUNTS = {'op': N_OP, 'machine': N_MACHINE, 'job': N_JOB}
EDGE_TYPES = [('job', 'contains', 'op'), ('op', 'belongs_to', 'job'),
              ('op', 'precedes', 'op'), ('op', 'machine_precedes', 'op'),
              ('op', 'on_machine', 'machine'), ('op', 'assigned_to', 'machine'),
              ('machine', 'can_process', 'op'), ('machine', 'processes', 'op')]
EDGE_COUNTS = {'contains': 10000, 'belongs_to': 10000, 'precedes': 10000,
               'machine_precedes': 10000, 'on_machine': 10000,
               'assigned_to': 10000, 'can_process': 30000, 'processes': 10000}

QOFF = {'op': 0, 'machine': N_OP, 'job': N_OP + N_MACHINE}
N_CAT = 10752
ROWS_PER_TILE = N_CAT // 16
PAD_DST = N_CAT - 1
E_TOT = sum(EDGE_COUNTS[r] for _, r, _ in EDGE_TYPES)
NW = 32
CH = 64
EPW_A = 3200
E_PAD = NW * EPW_A
EPT_B = E_PAD // 16
KR_ROWS = sum(NODE_COUNTS[st] for st, _, _ in EDGE_TYPES)
S_SEGS = 51200
PAD_SEG = S_SEGS - 1
SPR = S_SEGS // 8
S_RPT = SPR // 16


def _graph_norm(x, w, b, eps=1e-5):
    mean = jnp.mean(x)
    out = x - mean
    var = jnp.mean(out * out)
    return out / jnp.sqrt(var + eps) * w + b


_GDN = lax.GatherDimensionNumbers(offset_dims=(), collapsed_slice_dims=(0,),
                                  start_index_map=(0,))


def _lane_rot(v, r):
    idx = jnp.bitwise_and(lax.iota(jnp.int32, 16) + r, 15)
    return lax.gather(v, idx[:, None], _GDN, (1,),
                      mode=lax.GatherScatterMode.PROMISE_IN_BOUNDS)


def _allsum(v):
    for r in (8, 4, 2, 1):
        v = v + _lane_rot(v, r)
    return v

def _edge_a_body(q_hbm, kr_hbm, src_hbm, dst_hbm, seg_hbm, z_hbm, e_hbm, sp_hbm,
                 src_v, dst_v, seg_v, row_v, qrows, krows, e_flat, e128, s_tab,
                 sem_q, sem_k):
    cid = lax.axis_index('c')
    sid = lax.axis_index('s')
    wid = sid * 2 + cid
    iota = lax.iota(jnp.int32, 16)
    zero = jnp.zeros((16,), jnp.float32)
    pltpu.sync_copy(z_hbm, s_tab.at[pl.ds(sid * S_RPT, S_RPT)])
    plsc.subcore_barrier()

    def chunk(ci, _):
        base = wid * EPW_A + ci * CH
        pltpu.sync_copy(src_hbm.at[pl.ds(base, CH)], src_v)
        pltpu.sync_copy(dst_hbm.at[pl.ds(base, CH)], dst_v)
        pltpu.sync_copy(seg_hbm.at[pl.ds(base, CH)], seg_v.at[pl.ds(0, CH)])
        cp_q = pltpu.async_copy(q_hbm.at[dst_v], qrows, sem_q)
        cp_k = pltpu.async_copy(kr_hbm.at[src_v], krows, sem_k)
        for j in range(CH // 16):
            row_v[pl.ds(j * 16, 16)] = lax.shift_right_logical(
                seg_v[pl.ds(j * 16, 16)], 3)
        cp_q.wait()
        cp_k.wait()

        def edge(i, _):
            acc = jnp.zeros((16,), jnp.float32)
            for h in range(HEADS):
                q1 = qrows[i, pl.ds(h * 32, 16)]
                q2 = qrows[i, pl.ds(h * 32 + 16, 16)]
                k1 = krows[i, pl.ds(h * 32, 16)]
                k2 = krows[i, pl.ds(h * 32 + 16, 16)]
                sh = _allsum(q1 * k1 + q2 * k2)
                acc = jnp.where(iota == h, sh, acc)
            ev = jnp.exp(jnp.minimum(acc, 80.0))
            e_flat[pl.ds(i * 16, 16)] = ev
            for b in range(8):
                e128[i, pl.ds(b * 16, 16)] = zero
            blk = jnp.bitwise_and(seg_v[pl.ds(i, 16)][0], 7)
            e128[i, pl.ds(blk * 16, 16)] = ev
            return 0

        lax.fori_loop(0, CH, edge, 0)
        pltpu.sync_copy(e128, s_tab.at[row_v], add=True)
        pltpu.sync_copy(e_flat, e_hbm.at[pl.ds(base * 16, CH * 16)])
        return 0

    lax.fori_loop(0, EPW_A // CH, chunk, 0)
    plsc.subcore_barrier()
    pltpu.sync_copy(s_tab.at[pl.ds(sid * S_RPT, S_RPT)],
                    sp_hbm.at[cid, pl.ds(sid * S_RPT, S_RPT)])


@jax.jit
def _edge_pass_a(q_cat, kr_cat, srcg, dstg, segg):
    z128 = jnp.zeros((S_RPT, 128), jnp.float32)
    f = pl.kernel(
        _edge_a_body,
        out_type=[jax.ShapeDtypeStruct((E_PAD * 16,), jnp.float32),
                  jax.ShapeDtypeStruct((2, SPR, 128), jnp.float32)],
        mesh=plsc.VectorSubcoreMesh(core_axis_name='c', subcore_axis_name='s'),
        scratch_types=[
            pltpu.VMEM((CH,), jnp.int32),
            pltpu.VMEM((CH,), jnp.int32),
            pltpu.VMEM((CH + 16,), jnp.int32),
            pltpu.VMEM((CH,), jnp.int32),
            pltpu.VMEM((CH, HID), jnp.float32),
            pltpu.VMEM((CH, HID), jnp.float32),
            pltpu.VMEM((CH * 16,), jnp.float32),
            pltpu.VMEM((CH, 128), jnp.float32),
            pltpu.VMEM_SHARED((SPR, 128), jnp.float32),
            pltpu.SemaphoreType.DMA,
            pltpu.SemaphoreType.DMA,
        ],
    )
    return f(q_cat, kr_cat, srcg, dstg, segg, z128)


def _edge_a2_body(e_hbm, s_hbm, seg_hbm, w_hbm, seg_v, row_v, e_v, s128, sem_s):
    cid = lax.axis_index('c')
    sid = lax.axis_index('s')
    wid = sid * 2 + cid

    def chunk(ci, _):
        base = wid * EPW_A + ci * CH
        pltpu.sync_copy(seg_hbm.at[pl.ds(base, CH)], seg_v.at[pl.ds(0, CH)])
        pltpu.sync_copy(e_hbm.at[pl.ds(base * 16, CH * 16)], e_v)
        for j in range(CH // 16):
            row_v[pl.ds(j * 16, 16)] = lax.shift_right_logical(
                seg_v[pl.ds(j * 16, 16)], 3)
        pltpu.async_copy(s_hbm.at[row_v], s128, sem_s).wait()

        def edge(i, _):
            blk = jnp.bitwise_and(seg_v[pl.ds(i, 16)][0], 7)
            sv = s128[i, pl.ds(blk * 16, 16)]
            e_v[pl.ds(i * 16, 16)] = e_v[pl.ds(i * 16, 16)] / (sv + 1e-16)
            return 0

        lax.fori_loop(0, CH, edge, 0)
        pltpu.sync_copy(e_v, w_hbm.at[pl.ds(base * 16, CH * 16)])
        return 0

    lax.fori_loop(0, EPW_A // CH, chunk, 0)


@jax.jit
def _edge_pass_a2(e_all, s_tot, segg):
    f = pl.kernel(
        _edge_a2_body,
        out_type=[jax.ShapeDtypeStruct((E_PAD * 16,), jnp.float32)],
        mesh=plsc.VectorSubcoreMesh(core_axis_name='c', subcore_axis_name='s'),
        scratch_types=[
            pltpu.VMEM((CH + 16,), jnp.int32),
            pltpu.VMEM((CH,), jnp.int32),
            pltpu.VMEM((CH * 16,), jnp.float32),
            pltpu.VMEM((CH, 128), jnp.float32),
            pltpu.SemaphoreType.DMA,
        ],
    )
    return f(e_all, s_tot, segg)[0]


def _edge_b_body(vr_lo, vr_hi, src_hbm, dst_hbm, w_hbm, z_hbm,
                 lo_out, hi_out,
                 src_v, dst_v, vrows, w_v, m_buf, agg_tab, sem_v):
    cid = lax.axis_index('c')
    sid = lax.axis_index('s')
    iota = lax.iota(jnp.int32, 16)
    pltpu.sync_copy(z_hbm, agg_tab.at[pl.ds(sid * ROWS_PER_TILE, ROWS_PER_TILE)])
    plsc.subcore_barrier()

    def run(vr_hbm, out_hbm, hbase):
        def chunk(ci, _):
            base = sid * EPT_B + ci * CH
            pltpu.sync_copy(src_hbm.at[pl.ds(base, CH)], src_v)
            pltpu.sync_copy(dst_hbm.at[pl.ds(base, CH)], dst_v)
            cp_v = pltpu.async_copy(vr_hbm.at[src_v], vrows, sem_v)
            pltpu.sync_copy(w_hbm.at[pl.ds(base * 16, CH * 16)], w_v)
            cp_v.wait()

            def edge(i, _):
                w = w_v[pl.ds(i * 16, 16)]
                for jh in range(4):
                    wb = _allsum(jnp.where(iota == hbase + jh, w, 0.0))
                    c0 = jh * 32
                    m_buf[i, pl.ds(c0, 16)] = vrows[i, pl.ds(c0, 16)] * wb
                    m_buf[i, pl.ds(c0 + 16, 16)] = vrows[i, pl.ds(c0 + 16, 16)] * wb
                return 0

            lax.fori_loop(0, CH, edge, 0)
            pltpu.sync_copy(m_buf, agg_tab.at[dst_v], add=True)
            return 0

        lax.fori_loop(0, EPT_B // CH, chunk, 0)
        plsc.subcore_barrier()
        pltpu.sync_copy(agg_tab.at[pl.ds(sid * ROWS_PER_TILE, ROWS_PER_TILE)],
                        out_hbm.at[pl.ds(sid * ROWS_PER_TILE, ROWS_PER_TILE)])

    pl.when(cid == 0)(lambda: run(vr_lo, lo_out, 0))
    pl.when(cid == 1)(lambda: run(vr_hi, hi_out, 4))


@jax.jit
def _edge_pass_b(vr_lo, vr_hi, srcg, dstg, w_all):
    z128 = jnp.zeros((ROWS_PER_TILE, 128), jnp.float32)
    f = pl.kernel(
        _edge_b_body,
        out_type=[jax.ShapeDtypeStruct((N_CAT, 128), jnp.float32),
                  jax.ShapeDtypeStruct((N_CAT, 128), jnp.float32)],
        mesh=plsc.VectorSubcoreMesh(core_axis_name='c', subcore_axis_name='s'),
        scratch_types=[
            pltpu.VMEM((CH,), jnp.int32),
            pltpu.VMEM((CH,), jnp.int32),
            pltpu.VMEM((CH, 128), jnp.float32),
            pltpu.VMEM((CH * 16,), jnp.float32),
            pltpu.VMEM((CH, 128), jnp.float32),
            pltpu.VMEM_SHARED((N_CAT, 128), jnp.float32),
            pltpu.SemaphoreType.DMA,
        ],
    )
    return f(vr_lo, vr_hi, srcg, dstg, w_all, z128)


def _mlp_head_kernel(a_ref, w2_ref, b2_ref, w3_ref, b3_ref, w4_ref, b4_ref,
                     out_ref):
    h = jnp.tanh(a_ref[...])
    h = jnp.tanh(h @ w2_ref[...] + b2_ref[...])
    h = jnp.tanh(h @ w3_ref[...] + b3_ref[...])
    out_ref[...] = (h @ w4_ref[...] + b4_ref[...])


def _mlp_head(a, w2, b2, w3, b3, w4, b4):
    n = a.shape[0]
    blk = 1000
    return pl.pallas_call(
        _mlp_head_kernel,
        grid=(n // blk,),
        in_specs=[
            pl.BlockSpec((blk, HID), lambda i: (i, 0)),
            pl.BlockSpec((HID, HID), lambda i: (0, 0)),
            pl.BlockSpec((HID,), lambda i: (0,)),
            pl.BlockSpec((HID, 128), lambda i: (0, 0)),
            pl.BlockSpec((128,), lambda i: (0,)),
            pl.BlockSpec((128, 1), lambda i: (0, 0)),
            pl.BlockSpec((1,), lambda i: (0,)),
        ],
        out_specs=pl.BlockSpec((blk, 1), lambda i: (i, 0)),
        out_shape=jax.ShapeDtypeStruct((n, 1), jnp.float32),
    )(a, w2, b2, w3, b3, w4, b4)


def _hgt_layer(lp, x, srcg, dstg, segg):
    kd, qd, vd = {}, {}, {}
    for nt in NODE_TYPES:
        kqv = x[nt] @ lp['kqv_w_' + nt] + lp['kqv_b_' + nt]
        k, q, v = jnp.split(kqv, 3, axis=-1)
        kd[nt] = k.reshape(-1, HEADS, DH)
        qd[nt] = q.reshape(-1, HEADS, DH)
        vd[nt] = v.reshape(-1, HEADS, DH)

    kr_list, vr_list = [], []
    for st, rel, dt in EDGE_TYPES:
        a = lp['a_rel_' + rel] * (lp['p_rel_' + rel] / math.sqrt(DH))[:, None, None]
        kr = jnp.einsum('nhd,hde->nhe', kd[st], a).reshape(-1, HID)
        vr = jnp.einsum('nhd,hde->nhe', vd[st], lp['m_rel_' + rel]).reshape(-1, HID)
        kr_list.append(kr)
        vr_list.append(vr)
    kr_cat = jnp.concatenate(kr_list, axis=0)
    vr_cat = jnp.concatenate(vr_list, axis=0)
    q_cat = jnp.concatenate(
        [qd[nt].reshape(-1, HID) for nt in NODE_TYPES]
        + [jnp.zeros((N_CAT - QOFF['job'] - N_JOB, HID), jnp.float32)], axis=0)

    e_all, s_part = _edge_pass_a(q_cat, kr_cat, srcg, dstg, segg)
    s_tot = s_part[0] + s_part[1]
    w_all = _edge_pass_a2(e_all, s_tot, segg)
    out_lo, out_hi = _edge_pass_b(vr_cat[:, :128], vr_cat[:, 128:],
                                  srcg, dstg, w_all)
    agg_cat = jnp.concatenate([out_lo, out_hi], axis=1)

    out = {}
    for nt in NODE_TYPES:
        o = jax.nn.gelu(agg_cat[QOFF[nt]:QOFF[nt] + NODE_COUNTS[nt]])
        o = o @ lp['out_w_' + nt] + lp['out_b_' + nt]
        a = jax.nn.sigmoid(lp['skip_' + nt])
        out[nt] = a * o + (1.0 - a) * x[nt]
    return out


def kernel(params, op_x, machine_x, job_x, src_contains, dst_contains,
           src_belongs_to, dst_belongs_to, src_precedes, dst_precedes,
           src_machine_precedes, dst_machine_precedes, src_on_machine,
           dst_on_machine, src_assigned_to, dst_assigned_to, src_can_process,
           dst_can_process, src_processes, dst_processes, act_op, act_machine):
    edges = {'contains': (src_contains, dst_contains),
             'belongs_to': (src_belongs_to, dst_belongs_to),
             'precedes': (src_precedes, dst_precedes),
             'machine_precedes': (src_machine_precedes, dst_machine_precedes),
             'on_machine': (src_on_machine, dst_on_machine),
             'assigned_to': (src_assigned_to, dst_assigned_to),
             'can_process': (src_can_process, dst_can_process),
             'processes': (src_processes, dst_processes)}

    src_parts, dst_parts, seg_parts = [], [], []
    kr_off, s_off = 0, 0
    for st, rel, dt in EDGE_TYPES:
        s, d = edges[rel]
        src_parts.append(s + kr_off)
        dst_parts.append(d + QOFF[dt])
        seg_parts.append(d + s_off)
        kr_off += NODE_COUNTS[st]
        s_off += NODE_COUNTS[dt]
    srcg = jnp.concatenate(
        src_parts + [jnp.zeros((E_PAD - E_TOT,), jnp.int32)])
    dstg = jnp.concatenate(
        dst_parts + [jnp.full((E_PAD - E_TOT,), PAD_DST, jnp.int32)])
    segg = jnp.concatenate(
        seg_parts + [jnp.full((E_PAD - E_TOT,), PAD_SEG, jnp.int32)])

    feats = {'op': op_x, 'machine': machine_x, 'job': job_x}
    x = {}
    for nt in NODE_TYPES:
        h = feats[nt] @ params['emb_w_' + nt] + params['emb_b_' + nt]
        x[nt] = _graph_norm(h, params['norm0_w_' + nt], params['norm0_b_' + nt])
    for lp in params['layers']:
        res = x
        out = _hgt_layer(lp, x, srcg, dstg, segg)
        x = {nt: _graph_norm(out[nt] + res[nt], lp['ln_w_' + nt], lp['ln_b_' + nt])
             for nt in NODE_TYPES}

    g = jnp.concatenate([jnp.mean(x['op'], axis=0),
                         jnp.mean(x['machine'], axis=0),
                         jnp.mean(x['job'], axis=0)])
    w1 = params['mlp_w'][0]
    p_op = x['op'] @ w1[:HID]
    p_m = x['machine'] @ w1[HID:2 * HID]
    cg = g @ w1[2 * HID:] + params['mlp_b'][0]
    a = p_op[act_op] + p_m[act_machine] + cg[None, :]
    out = _mlp_head(a, params['mlp_w'][1], params['mlp_b'][1],
                    params['mlp_w'][2], params['mlp_b'][2],
                    params['mlp_w'][3], params['mlp_b'][3])
    return out[:, 0]


# speedup vs baseline: 8.7905x; 1.1694x over previous
import math
import functools

import jax
import jax.numpy as jnp
from jax import lax
from jax.experimental import pallas as pl
from jax.experimental.pallas import tpu as pltpu, tpu_sc as plsc

N_OP, N_MACHINE, N_JOB = 10000, 200, 500
HID, HEADS = 256, 8
DH = HID // HEADS
NODE_TYPES = ['op', 'machine', 'job']
NODE_COUNTS = {'op': N_OP, 'machine': N_MACHINE, 'job': N_JOB}
EDGE_TYPES = [('job', 'contains', 'op'), ('op', 'belongs_to', 'job'),
              ('op', 'precedes', 'op'), ('op', 'machine_precedes', 'op'),
              ('op', 'on_machine', 'machine'), ('op', 'assigned_to', 'machine'),
              ('machine', 'can_process', 'op'), ('machine', 'processes', 'op')]
EDGE_COUNTS = {'contains': 10000, 'belongs_to': 10000, 'precedes': 10000,
               'machine_precedes': 10000, 'on_machine': 10000,
               'assigned_to': 10000, 'can_process': 30000, 'processes': 10000}

QOFF = {'op': 0, 'machine': N_OP, 'job': N_OP + N_MACHINE}
N_CAT = 10752
ROWS_PER_TILE = N_CAT // 16
PAD_DST = N_CAT - 1
E_TOT = sum(EDGE_COUNTS[r] for _, r, _ in EDGE_TYPES)
NW = 32
CH = 64
EPW_A = 3200
E_PAD = NW * EPW_A
EPT_B = E_PAD // 16
KR_ROWS = sum(NODE_COUNTS[st] for st, _, _ in EDGE_TYPES)
S_SEGS = 51200
PAD_SEG = S_SEGS - 1
SPR = S_SEGS // 8
S_RPT = SPR // 16


def _graph_norm(x, w, b, eps=1e-5):
    mean = jnp.mean(x)
    out = x - mean
    var = jnp.mean(out * out)
    return out / jnp.sqrt(var + eps) * w + b


_GDN = lax.GatherDimensionNumbers(offset_dims=(), collapsed_slice_dims=(0,),
                                  start_index_map=(0,))


def _lane_rot(v, r):
    idx = jnp.bitwise_and(lax.iota(jnp.int32, 16) + r, 15)
    return lax.gather(v, idx[:, None], _GDN, (1,),
                      mode=lax.GatherScatterMode.PROMISE_IN_BOUNDS)


def _allsum(v):
    for r in (8, 4, 2, 1):
        v = v + _lane_rot(v, r)
    return v

def _edge_a_body(q_hbm, kr_hbm, src_hbm, dst_hbm, seg_hbm, z_hbm, e_hbm, sp_hbm,
                 src_v, dst_v, seg_v, row_v, qrows, krows, e_flat, e128, s_tab,
                 sem_q, sem_k):
    cid = lax.axis_index('c')
    sid = lax.axis_index('s')
    wid = sid * 2 + cid
    iota = lax.iota(jnp.int32, 16)
    zero = jnp.zeros((16,), jnp.float32)
    pltpu.sync_copy(z_hbm, s_tab.at[pl.ds(sid * S_RPT, S_RPT)])
    plsc.subcore_barrier()

    def chunk(ci, _):
        base = wid * EPW_A + ci * CH
        pltpu.sync_copy(src_hbm.at[pl.ds(base, CH)], src_v)
        pltpu.sync_copy(dst_hbm.at[pl.ds(base, CH)], dst_v)
        pltpu.sync_copy(seg_hbm.at[pl.ds(base, CH)], seg_v.at[pl.ds(0, CH)])
        cp_q = pltpu.async_copy(q_hbm.at[dst_v], qrows, sem_q)
        cp_k = pltpu.async_copy(kr_hbm.at[src_v], krows, sem_k)
        for j in range(CH // 16):
            row_v[pl.ds(j * 16, 16)] = lax.shift_right_logical(
                seg_v[pl.ds(j * 16, 16)], 3)
        cp_q.wait()
        cp_k.wait()

        @plsc.parallel_loop(0, CH, unroll=4)
        def edge(i):
            acc = jnp.zeros((16,), jnp.float32)
            for h in range(HEADS):
                q1 = qrows[i, pl.ds(h * 32, 16)]
                q2 = qrows[i, pl.ds(h * 32 + 16, 16)]
                k1 = krows[i, pl.ds(h * 32, 16)]
                k2 = krows[i, pl.ds(h * 32 + 16, 16)]
                sh = _allsum(q1 * k1 + q2 * k2)
                acc = jnp.where(iota == h, sh, acc)
            ev = jnp.exp(jnp.minimum(acc, 80.0))
            e_flat[pl.ds(i * 16, 16)] = ev
            for b in range(8):
                e128[i, pl.ds(b * 16, 16)] = zero
            blk = jnp.bitwise_and(seg_v[pl.ds(i, 16)][0], 7)
            e128[i, pl.ds(blk * 16, 16)] = ev
        pltpu.sync_copy(e128, s_tab.at[row_v], add=True)
        pltpu.sync_copy(e_flat, e_hbm.at[pl.ds(base * 16, CH * 16)])
        return 0

    lax.fori_loop(0, EPW_A // CH, chunk, 0)
    plsc.subcore_barrier()
    pltpu.sync_copy(s_tab.at[pl.ds(sid * S_RPT, S_RPT)],
                    sp_hbm.at[cid, pl.ds(sid * S_RPT, S_RPT)])


@jax.jit
def _edge_pass_a(q_cat, kr_cat, srcg, dstg, segg):
    z128 = jnp.zeros((S_RPT, 128), jnp.float32)
    f = pl.kernel(
        _edge_a_body,
        out_type=[jax.ShapeDtypeStruct((E_PAD * 16,), jnp.float32),
                  jax.ShapeDtypeStruct((2, SPR, 128), jnp.float32)],
        mesh=plsc.VectorSubcoreMesh(core_axis_name='c', subcore_axis_name='s'),
        scratch_types=[
            pltpu.VMEM((CH,), jnp.int32),
            pltpu.VMEM((CH,), jnp.int32),
            pltpu.VMEM((CH + 16,), jnp.int32),
            pltpu.VMEM((CH,), jnp.int32),
            pltpu.VMEM((CH, HID), jnp.float32),
            pltpu.VMEM((CH, HID), jnp.float32),
            pltpu.VMEM((CH * 16,), jnp.float32),
            pltpu.VMEM((CH, 128), jnp.float32),
            pltpu.VMEM_SHARED((SPR, 128), jnp.float32),
            pltpu.SemaphoreType.DMA,
            pltpu.SemaphoreType.DMA,
        ],
    )
    return f(q_cat, kr_cat, srcg, dstg, segg, z128)


def _edge_a2_body(e_hbm, s_hbm, seg_hbm, w_hbm, seg_v, row_v, e_v, s128, sem_s):
    cid = lax.axis_index('c')
    sid = lax.axis_index('s')
    wid = sid * 2 + cid

    def chunk(ci, _):
        base = wid * EPW_A + ci * CH
        pltpu.sync_copy(seg_hbm.at[pl.ds(base, CH)], seg_v.at[pl.ds(0, CH)])
        pltpu.sync_copy(e_hbm.at[pl.ds(base * 16, CH * 16)], e_v)
        for j in range(CH // 16):
            row_v[pl.ds(j * 16, 16)] = lax.shift_right_logical(
                seg_v[pl.ds(j * 16, 16)], 3)
        pltpu.async_copy(s_hbm.at[row_v], s128, sem_s).wait()

        @plsc.parallel_loop(0, CH, unroll=8)
        def edge(i):
            blk = jnp.bitwise_and(seg_v[pl.ds(i, 16)][0], 7)
            sv = s128[i, pl.ds(blk * 16, 16)]
            e_v[pl.ds(i * 16, 16)] = e_v[pl.ds(i * 16, 16)] / (sv + 1e-16)
        pltpu.sync_copy(e_v, w_hbm.at[pl.ds(base * 16, CH * 16)])
        return 0

    lax.fori_loop(0, EPW_A // CH, chunk, 0)


@jax.jit
def _edge_pass_a2(e_all, s_tot, segg):
    f = pl.kernel(
        _edge_a2_body,
        out_type=[jax.ShapeDtypeStruct((E_PAD * 16,), jnp.float32)],
        mesh=plsc.VectorSubcoreMesh(core_axis_name='c', subcore_axis_name='s'),
        scratch_types=[
            pltpu.VMEM((CH + 16,), jnp.int32),
            pltpu.VMEM((CH,), jnp.int32),
            pltpu.VMEM((CH * 16,), jnp.float32),
            pltpu.VMEM((CH, 128), jnp.float32),
            pltpu.SemaphoreType.DMA,
        ],
    )
    return f(e_all, s_tot, segg)[0]


def _edge_b_body(vr_lo, vr_hi, src_hbm, dst_hbm, w_hbm, z_hbm,
                 lo_out, hi_out,
                 src_v, dst_v, vrows, w_v, m_buf, agg_tab, sem_v):
    cid = lax.axis_index('c')
    sid = lax.axis_index('s')
    iota = lax.iota(jnp.int32, 16)
    pltpu.sync_copy(z_hbm, agg_tab.at[pl.ds(sid * ROWS_PER_TILE, ROWS_PER_TILE)])
    plsc.subcore_barrier()

    def run(vr_hbm, out_hbm, hbase):
        def chunk(ci, _):
            base = sid * EPT_B + ci * CH
            pltpu.sync_copy(src_hbm.at[pl.ds(base, CH)], src_v)
            pltpu.sync_copy(dst_hbm.at[pl.ds(base, CH)], dst_v)
            cp_v = pltpu.async_copy(vr_hbm.at[src_v], vrows, sem_v)
            pltpu.sync_copy(w_hbm.at[pl.ds(base * 16, CH * 16)], w_v)
            cp_v.wait()

            @plsc.parallel_loop(0, CH, unroll=4)
            def edge(i):
                w = w_v[pl.ds(i * 16, 16)]
                for jh in range(4):
                    wb = _allsum(jnp.where(iota == hbase + jh, w, 0.0))
                    c0 = jh * 32
                    m_buf[i, pl.ds(c0, 16)] = vrows[i, pl.ds(c0, 16)] * wb
                    m_buf[i, pl.ds(c0 + 16, 16)] = vrows[i, pl.ds(c0 + 16, 16)] * wb
            pltpu.sync_copy(m_buf, agg_tab.at[dst_v], add=True)
            return 0

        lax.fori_loop(0, EPT_B // CH, chunk, 0)
        plsc.subcore_barrier()
        pltpu.sync_copy(agg_tab.at[pl.ds(sid * ROWS_PER_TILE, ROWS_PER_TILE)],
                        out_hbm.at[pl.ds(sid * ROWS_PER_TILE, ROWS_PER_TILE)])

    pl.when(cid == 0)(lambda: run(vr_lo, lo_out, 0))
    pl.when(cid == 1)(lambda: run(vr_hi, hi_out, 4))


@jax.jit
def _edge_pass_b(vr_lo, vr_hi, srcg, dstg, w_all):
    z128 = jnp.zeros((ROWS_PER_TILE, 128), jnp.float32)
    f = pl.kernel(
        _edge_b_body,
        out_type=[jax.ShapeDtypeStruct((N_CAT, 128), jnp.float32),
                  jax.ShapeDtypeStruct((N_CAT, 128), jnp.float32)],
        mesh=plsc.VectorSubcoreMesh(core_axis_name='c', subcore_axis_name='s'),
        scratch_types=[
            pltpu.VMEM((CH,), jnp.int32),
            pltpu.VMEM((CH,), jnp.int32),
            pltpu.VMEM((CH, 128), jnp.float32),
            pltpu.VMEM((CH * 16,), jnp.float32),
            pltpu.VMEM((CH, 128), jnp.float32),
            pltpu.VMEM_SHARED((N_CAT, 128), jnp.float32),
            pltpu.SemaphoreType.DMA,
        ],
    )
    return f(vr_lo, vr_hi, srcg, dstg, w_all, z128)


def _mlp_head_kernel(a_ref, w2_ref, b2_ref, w3_ref, b3_ref, w4_ref, b4_ref,
                     out_ref):
    h = jnp.tanh(a_ref[...])
    h = jnp.tanh(h @ w2_ref[...] + b2_ref[...])
    h = jnp.tanh(h @ w3_ref[...] + b3_ref[...])
    out_ref[...] = (h @ w4_ref[...] + b4_ref[...])


def _mlp_head(a, w2, b2, w3, b3, w4, b4):
    n = a.shape[0]
    blk = 1000
    return pl.pallas_call(
        _mlp_head_kernel,
        grid=(n // blk,),
        in_specs=[
            pl.BlockSpec((blk, HID), lambda i: (i, 0)),
            pl.BlockSpec((HID, HID), lambda i: (0, 0)),
            pl.BlockSpec((HID,), lambda i: (0,)),
            pl.BlockSpec((HID, 128), lambda i: (0, 0)),
            pl.BlockSpec((128,), lambda i: (0,)),
            pl.BlockSpec((128, 1), lambda i: (0, 0)),
            pl.BlockSpec((1,), lambda i: (0,)),
        ],
        out_specs=pl.BlockSpec((blk, 1), lambda i: (i, 0)),
        out_shape=jax.ShapeDtypeStruct((n, 1), jnp.float32),
    )(a, w2, b2, w3, b3, w4, b4)


def _hgt_layer(lp, x, srcg, dstg, segg):
    kd, qd, vd = {}, {}, {}
    for nt in NODE_TYPES:
        kqv = x[nt] @ lp['kqv_w_' + nt] + lp['kqv_b_' + nt]
        k, q, v = jnp.split(kqv, 3, axis=-1)
        kd[nt] = k.reshape(-1, HEADS, DH)
        qd[nt] = q.reshape(-1, HEADS, DH)
        vd[nt] = v.reshape(-1, HEADS, DH)

    kr_list, vr_list = [], []
    for st, rel, dt in EDGE_TYPES:
        a = lp['a_rel_' + rel] * (lp['p_rel_' + rel] / math.sqrt(DH))[:, None, None]
        kr = jnp.einsum('nhd,hde->nhe', kd[st], a).reshape(-1, HID)
        vr = jnp.einsum('nhd,hde->nhe', vd[st], lp['m_rel_' + rel]).reshape(-1, HID)
        kr_list.append(kr)
        vr_list.append(vr)
    kr_cat = jnp.concatenate(kr_list, axis=0)
    vr_cat = jnp.concatenate(vr_list, axis=0)
    q_cat = jnp.concatenate(
        [qd[nt].reshape(-1, HID) for nt in NODE_TYPES]
        + [jnp.zeros((N_CAT - QOFF['job'] - N_JOB, HID), jnp.float32)], axis=0)

    e_all, s_part = _edge_pass_a(q_cat, kr_cat, srcg, dstg, segg)
    s_tot = s_part[0] + s_part[1]
    w_all = _edge_pass_a2(e_all, s_tot, segg)
    out_lo, out_hi = _edge_pass_b(vr_cat[:, :128], vr_cat[:, 128:],
                                  srcg, dstg, w_all)
    agg_cat = jnp.concatenate([out_lo, out_hi], axis=1)

    out = {}
    for nt in NODE_TYPES:
        o = jax.nn.gelu(agg_cat[QOFF[nt]:QOFF[nt] + NODE_COUNTS[nt]])
        o = o @ lp['out_w_' + nt] + lp['out_b_' + nt]
        a = jax.nn.sigmoid(lp['skip_' + nt])
        out[nt] = a * o + (1.0 - a) * x[nt]
    return out


def kernel(params, op_x, machine_x, job_x, src_contains, dst_contains,
           src_belongs_to, dst_belongs_to, src_precedes, dst_precedes,
           src_machine_precedes, dst_machine_precedes, src_on_machine,
           dst_on_machine, src_assigned_to, dst_assigned_to, src_can_process,
           dst_can_process, src_processes, dst_processes, act_op, act_machine):
    edges = {'contains': (src_contains, dst_contains),
             'belongs_to': (src_belongs_to, dst_belongs_to),
             'precedes': (src_precedes, dst_precedes),
             'machine_precedes': (src_machine_precedes, dst_machine_precedes),
             'on_machine': (src_on_machine, dst_on_machine),
             'assigned_to': (src_assigned_to, dst_assigned_to),
             'can_process': (src_can_process, dst_can_process),
             'processes': (src_processes, dst_processes)}

    src_parts, dst_parts, seg_parts = [], [], []
    kr_off, s_off = 0, 0
    for st, rel, dt in EDGE_TYPES:
        s, d = edges[rel]
        src_parts.append(s + kr_off)
        dst_parts.append(d + QOFF[dt])
        seg_parts.append(d + s_off)
        kr_off += NODE_COUNTS[st]
        s_off += NODE_COUNTS[dt]
    srcg = jnp.concatenate(
        src_parts + [jnp.zeros((E_PAD - E_TOT,), jnp.int32)])
    dstg = jnp.concatenate(
        dst_parts + [jnp.full((E_PAD - E_TOT,), PAD_DST, jnp.int32)])
    segg = jnp.concatenate(
        seg_parts + [jnp.full((E_PAD - E_TOT,), PAD_SEG, jnp.int32)])

    feats = {'op': op_x, 'machine': machine_x, 'job': job_x}
    x = {}
    for nt in NODE_TYPES:
        h = feats[nt] @ params['emb_w_' + nt] + params['emb_b_' + nt]
        x[nt] = _graph_norm(h, params['norm0_w_' + nt], params['norm0_b_' + nt])
    for lp in params['layers']:
        res = x
        out = _hgt_layer(lp, x, srcg, dstg, segg)
        x = {nt: _graph_norm(out[nt] + res[nt], lp['ln_w_' + nt], lp['ln_b_' + nt])
             for nt in NODE_TYPES}

    g = jnp.concatenate([jnp.mean(x['op'], axis=0),
                         jnp.mean(x['machine'], axis=0),
                         jnp.mean(x['job'], axis=0)])
    w1 = params['mlp_w'][0]
    p_op = x['op'] @ w1[:HID]
    p_m = x['machine'] @ w1[HID:2 * HID]
    cg = g @ w1[2 * HID:] + params['mlp_b'][0]
    a = p_op[act_op] + p_m[act_machine] + cg[None, :]
    out = _mlp_head(a, params['mlp_w'][1], params['mlp_b'][1],
                    params['mlp_w'][2], params['mlp_b'][2],
                    params['mlp_w'][3], params['mlp_b'][3])
    return out[:, 0]
